# baseline (speedup 1.0000x reference)
import functools

import jax
import jax.numpy as jnp
from jax import lax
from jax.experimental import pallas as pl
from jax.experimental.pallas import tpu as pltpu
from jax.experimental.pallas import tpu_sc as plsc

NC = 2
NS = 16
NW = NC * NS
L = 16
CH = 128
BIG = 1e30


def _cdiv(a, b):
    return (a + b - 1) // b


def _rsqrt_newton(s):
    i = plsc.bitcast(s, jnp.int32)
    i = jnp.int32(0x5F3759DF) - lax.shift_right_logical(i, 1)
    y = plsc.bitcast(i, jnp.float32)
    for _ in range(3):
        y = y * (jnp.float32(1.5) - jnp.float32(0.5) * s * y * y)
    return y


def _dist16(tA, tB, rows):
    r3 = rows * 3
    dx = plsc.load_gather(tA, [r3]) - plsc.load_gather(tB, [r3])
    dy = plsc.load_gather(tA, [r3 + 1]) - plsc.load_gather(tB, [r3 + 1])
    dz = plsc.load_gather(tA, [r3 + 2]) - plsc.load_gather(tB, [r3 + 2])
    return dx * dx + dy * dy + dz * dz


def _make_pass1(B, TM, TMP, TN, TNP):
    mesh = plsc.VectorSubcoreMesh(core_axis_name="c", subcore_axis_name="s")
    n_mchunk = TMP // CH
    n_nchunk = TNP // CH

    @functools.partial(
        pl.kernel,
        mesh=mesh,
        compiler_params=pltpu.CompilerParams(use_tc_tiling_on_sc=False, needs_layout_passes=False),
        out_type=[
            jax.ShapeDtypeStruct((NW, 8 * L), jnp.float32),
            jax.ShapeDtypeStruct((B, NW, TNP), jnp.float32),
        ],
        scratch_types=[
            pltpu.VMEM((3 * max(TMP, TNP),), jnp.int32),
            pltpu.VMEM((3 * max(TMP, TNP),), jnp.int32),
            pltpu.VMEM((3 * CH,), jnp.float32),
            pltpu.VMEM((3 * CH,), jnp.float32),
            pltpu.VMEM((3 * CH,), jnp.float32),
            pltpu.VMEM((3 * CH,), jnp.float32),
            pltpu.VMEM((TNP,), jnp.float32),
            pltpu.VMEM((8 * L,), jnp.float32),
            pltpu.SemaphoreType.DMA,
            pltpu.SemaphoreType.DMA,
            pltpu.SemaphoreType.DMA,
            pltpu.SemaphoreType.DMA,
        ],
    )
    def pass1(tabA, tabB, mA, mB, nmA, nmB, part_out, dist_out,
              idxA_v, idxB_v, rA0, rB0, rA1, rB1, dist_v, part_v,
              sA0, sB0, sA1, sB1):
        wid = lax.axis_index("s") * NC + lax.axis_index("c")
        lane = lax.iota(jnp.int32, L)
        zeros = jnp.zeros((L,), jnp.float32)

        def issue(c, rA, rB, sA, sB):
            for k in range(3):
                pltpu.async_copy(
                    tabA.at[idxA_v.at[pl.ds((3 * c + k) * CH, CH)]],
                    rA.at[pl.ds(k * CH, CH)], sA)
                pltpu.async_copy(
                    tabB.at[idxB_v.at[pl.ds((3 * c + k) * CH, CH)]],
                    rB.at[pl.ds(k * CH, CH)], sB)

        def drain(buf, sem):
            pltpu.make_async_copy(tabA.at[pl.ds(0, 3 * CH)], buf, sem).wait()

        pltpu.sync_copy(mA.at[wid], idxA_v.at[pl.ds(0, 3 * TMP)])
        pltpu.sync_copy(mB.at[wid], idxB_v.at[pl.ds(0, 3 * TMP)])

        def msum(c, rA, rB, acc):
            for j in range(CH // L):
                rows = j * L + lane
                s = _dist16(rA, rB, rows)
                valid = (c * CH + j * L + lane) < TM
                acc = acc + jnp.where(valid, s, jnp.float32(0.0))
            return acc

        issue(0, rA0, rB0, sA0, sB0)

        def mchunk2(c2, acc):
            c = 2 * c2
            issue(c + 1, rA1, rB1, sA1, sB1)
            drain(rA0, sA0)
            drain(rB0, sB0)
            acc = msum(c, rA0, rB0, acc)

            @pl.when(c + 2 < n_mchunk)
            def _():
                issue(c + 2, rA0, rB0, sA0, sB0)

            drain(rA1, sA1)
            drain(rB1, sB1)
            acc = msum(c + 1, rA1, rB1, acc)
            return acc

        macc = lax.fori_loop(0, n_mchunk // 2, mchunk2, zeros)
        part_v[pl.ds(0, L)] = macc
        for r in range(5, 8):
            part_v[pl.ds(r * L, L)] = zeros

        def ndist(c, rA, rB, acc):
            for j in range(CH // L):
                rows = j * L + lane
                s = _dist16(rA, rB, rows)
                d = s * _rsqrt_newton(s)
                d = jnp.where(s > jnp.float32(0.0), d, jnp.float32(0.0))
                valid = (c * CH + j * L + lane) < TN
                dist_v[pl.ds(c * CH + j * L, L)] = jnp.where(
                    valid, d, jnp.float32(BIG))
                acc = acc + jnp.where(valid, d, jnp.float32(0.0))
            return acc

        for b in range(B):
            pltpu.sync_copy(nmA.at[b, wid], idxA_v)
            pltpu.sync_copy(nmB.at[b, wid], idxB_v)
            issue(0, rA0, rB0, sA0, sB0)

            def nchunk2(c2, acc):
                c = 2 * c2
                issue(c + 1, rA1, rB1, sA1, sB1)
                drain(rA0, sA0)
                drain(rB0, sB0)
                acc = ndist(c, rA0, rB0, acc)

                @pl.when(c + 2 < n_nchunk)
                def _():
                    issue(c + 2, rA0, rB0, sA0, sB0)

                drain(rA1, sA1)
                drain(rB1, sB1)
                acc = ndist(c + 1, rA1, rB1, acc)
                return acc

            nacc = lax.fori_loop(0, n_nchunk // 2, nchunk2, zeros)
            part_v[pl.ds((1 + b) * L, L)] = nacc
            pltpu.sync_copy(dist_v, dist_out.at[b, wid])

        pltpu.sync_copy(part_v, part_out.at[wid])

    return pass1


def _make_pass2(B, TNP):
    mesh = plsc.VectorSubcoreMesh(core_axis_name="c", subcore_axis_name="s")
    n_chunk = TNP // L

    @functools.partial(
        pl.kernel,
        mesh=mesh,
        compiler_params=pltpu.CompilerParams(use_tc_tiling_on_sc=False, needs_layout_passes=False),
        out_type=jax.ShapeDtypeStruct((NW, 8 * L), jnp.float32),
        scratch_types=[
            pltpu.VMEM((TNP,), jnp.float32),
            pltpu.VMEM((L,), jnp.float32),
            pltpu.VMEM((8 * L,), jnp.float32),
        ],
    )
    def pass2(dist, mrep, part_out, dist_v, m_v, part_v):
        wid = lax.axis_index("s") * NC + lax.axis_index("c")
        zeros = jnp.zeros((L,), jnp.float32)
        for b in range(B):
            pltpu.sync_copy(mrep.at[b], m_v)
            pltpu.sync_copy(dist.at[b, wid], dist_v)
            m = m_v[...]

            def chunk(k, carry):
                sacc, cacc = carry
                d = dist_v[pl.ds(k * L, L)]
                h = jnp.maximum(m - d, jnp.float32(0.0))
                h2 = h * h
                return (sacc + h2,
                        cacc + jnp.where(h2 > jnp.float32(0.0),
                                         jnp.float32(1.0), jnp.float32(0.0)))

            sacc, cacc = lax.fori_loop(0, n_chunk, chunk, (zeros, zeros))
            part_v[pl.ds(b * L, L)] = sacc
            part_v[pl.ds((4 + b) * L, L)] = cacc
        pltpu.sync_copy(part_v, part_out.at[wid])

    return pass2


def kernel(outA, outB, matchA, matchB, nonMatchA, nonMatchB, hardNegative,
           device):
    B, N, D = outA.shape
    nM = matchA.shape[1]
    nNM = nonMatchA.shape[1]
    TM = (B * nM) // NW
    TMP = _cdiv(TM, 2 * CH) * 2 * CH
    TN = nNM // NW
    TNP = _cdiv(TN, 2 * CH) * 2 * CH

    tabA = outA.transpose(2, 0, 1).reshape(B * N * D)
    tabB = outB.transpose(2, 0, 1).reshape(B * N * D)
    offs = (jnp.arange(B, dtype=jnp.int32) * N)[:, None]
    c3 = jnp.arange(D, dtype=jnp.int32) * (B * N)

    def expand3(idx):
        return (idx[..., None] + c3).reshape(*idx.shape[:-1], -1)

    mA = expand3(jnp.pad((matchA.astype(jnp.int32) + offs).reshape(NW, TM),
                         ((0, 0), (0, TMP - TM))))
    mB = expand3(jnp.pad((matchB.astype(jnp.int32) + offs).reshape(NW, TM),
                         ((0, 0), (0, TMP - TM))))
    nmA = expand3(jnp.pad(
        (nonMatchA.astype(jnp.int32) + offs).reshape(B, NW, TN),
        ((0, 0), (0, 0), (0, TNP - TN))))
    nmB = expand3(jnp.pad(
        (nonMatchB.astype(jnp.int32) + offs).reshape(B, NW, TN),
        ((0, 0), (0, 0), (0, TNP - TN))))

    part1, dist = _make_pass1(B, TM, TMP, TN, TNP)(tabA, tabB, mA, mB,
                                                   nmA, nmB)

    matchLossSum = part1[:, 0:L].sum() / nM
    distSum = part1[:, L:(1 + B) * L].reshape(NW, B, L).sum(axis=(0, 2))
    meanDist = distSum / nNM
    mrep = jnp.broadcast_to(meanDist[:, None], (B, L))

    part2 = _make_pass2(B, TNP)(dist, mrep)
    nmSum = part2[:, 0:B * L].reshape(NW, B, L).sum(axis=(0, 2))
    cnt = part2[:, 4 * L:(4 + B) * L].reshape(NW, B, L).sum(axis=(0, 2))

    denom = jnp.where(cnt == 0, jnp.float32(nNM), cnt)
    hard = nmSum / denom
    soft = nmSum / nNM
    nmLoss = jnp.where(jnp.asarray(hardNegative) != 0, hard, soft)
    nonMatchLossSum = nmLoss.sum()
    contrastiveLossSum = matchLossSum + nonMatchLossSum
    return (contrastiveLossSum.astype(jnp.float32),
            matchLossSum.astype(jnp.float32),
            nonMatchLossSum.astype(jnp.float32))


# speedup vs baseline: 1.0144x; 1.0144x over previous
import functools

import jax
import jax.numpy as jnp
from jax import lax
from jax.experimental import pallas as pl
from jax.experimental.pallas import tpu as pltpu
from jax.experimental.pallas import tpu_sc as plsc

NC = 2
NS = 16
NW = NC * NS
L = 16
CH = 128
BIG = 1e30


def _cdiv(a, b):
    return (a + b - 1) // b


def _rsqrt_newton(s):
    i = plsc.bitcast(s, jnp.int32)
    i = jnp.int32(0x5F3759DF) - lax.shift_right_logical(i, 1)
    y = plsc.bitcast(i, jnp.float32)
    for _ in range(3):
        y = y * (jnp.float32(1.5) - jnp.float32(0.5) * s * y * y)
    return y


def _dist16(tA, tB, rows):
    r3 = rows * 3
    dx = plsc.load_gather(tA, [r3]) - plsc.load_gather(tB, [r3])
    dy = plsc.load_gather(tA, [r3 + 1]) - plsc.load_gather(tB, [r3 + 1])
    dz = plsc.load_gather(tA, [r3 + 2]) - plsc.load_gather(tB, [r3 + 2])
    return dx * dx + dy * dy + dz * dz


def _make_kernel(B, nNM, TM, TMP, TN, TNP):
    mesh = plsc.VectorSubcoreMesh(core_axis_name="c", subcore_axis_name="s")
    n_mchunk = TMP // CH
    n_nchunk = TNP // CH
    n_local = B // NC

    @functools.partial(
        pl.kernel,
        mesh=mesh,
        compiler_params=pltpu.CompilerParams(use_tc_tiling_on_sc=False,
                                             needs_layout_passes=False),
        out_type=jax.ShapeDtypeStruct((NW, 8 * L), jnp.float32),
        scratch_types=[
            pltpu.VMEM((3 * max(TMP, TNP),), jnp.int32),
            pltpu.VMEM((3 * max(TMP, TNP),), jnp.int32),
            pltpu.VMEM((3 * CH,), jnp.float32),
            pltpu.VMEM((3 * CH,), jnp.float32),
            pltpu.VMEM((n_local, TNP), jnp.float32),
            pltpu.VMEM((n_local * L,), jnp.float32),
            pltpu.VMEM((NS, n_local * L), jnp.float32),
            pltpu.VMEM((8 * L,), jnp.float32),
            pltpu.VMEM_SHARED((NS, n_local * L), jnp.float32),
            pltpu.SemaphoreType.DMA,
            pltpu.SemaphoreType.DMA,
        ],
    )
    def body(tabA, tabB, mA, mB, nmA, nmB, part_out,
             idxA_v, idxB_v, rA, rB, dist_v, stage_v, allsums_v, part_v,
             shared, semA, semB):
        cid = lax.axis_index("c")
        sid = lax.axis_index("s")
        wid = cid * NS + sid
        lane = lax.iota(jnp.int32, L)
        zeros = jnp.zeros((L,), jnp.float32)

        def gather_chunk(c):
            cps = []
            for k in range(3):
                cps.append(pltpu.async_copy(
                    tabA.at[idxA_v.at[pl.ds((3 * c + k) * CH, CH)]],
                    rA.at[pl.ds(k * CH, CH)], semA))
                cps.append(pltpu.async_copy(
                    tabB.at[idxB_v.at[pl.ds((3 * c + k) * CH, CH)]],
                    rB.at[pl.ds(k * CH, CH)], semB))
            for cp in cps:
                cp.wait()

        pltpu.sync_copy(mA.at[wid], idxA_v.at[pl.ds(0, 3 * TMP)])
        pltpu.sync_copy(mB.at[wid], idxB_v.at[pl.ds(0, 3 * TMP)])

        def mchunk(c, acc):
            gather_chunk(c)
            for j in range(CH // L):
                rows = j * L + lane
                s = _dist16(rA, rB, rows)
                valid = (c * CH + j * L + lane) < TM
                acc = acc + jnp.where(valid, s, jnp.float32(0.0))
            return acc

        macc = lax.fori_loop(0, n_mchunk, mchunk, zeros)
        part_v[pl.ds(0, L)] = macc
        for r in range(5, 8):
            part_v[pl.ds(r * L, L)] = zeros

        for lb in range(n_local):
            pltpu.sync_copy(nmA.at[n_local * cid + lb, sid], idxA_v)
            pltpu.sync_copy(nmB.at[n_local * cid + lb, sid], idxB_v)

            def nchunk(c, acc):
                gather_chunk(c)
                for j in range(CH // L):
                    rows = j * L + lane
                    s = _dist16(rA, rB, rows)
                    d = s * _rsqrt_newton(s)
                    d = jnp.where(s > jnp.float32(0.0), d, jnp.float32(0.0))
                    valid = (c * CH + j * L + lane) < TN
                    dist_v[lb, pl.ds(c * CH + j * L, L)] = jnp.where(
                        valid, d, jnp.float32(BIG))
                    acc = acc + jnp.where(valid, d, jnp.float32(0.0))
                return acc

            nacc = lax.fori_loop(0, n_nchunk, nchunk, zeros)
            stage_v[pl.ds(lb * L, L)] = nacc

        pltpu.sync_copy(stage_v, shared.at[sid])
        plsc.subcore_barrier()
        pltpu.sync_copy(shared, allsums_v)

        for lb in range(n_local):
            dacc = zeros
            for t in range(NS):
                dacc = dacc + allsums_v[t, pl.ds(lb * L, L)]
            mean = jnp.sum(dacc) * jnp.float32(1.0 / nNM)
            mvec = jnp.broadcast_to(mean, (L,))

            def hchunk(k, carry):
                hs, hc = carry
                d = dist_v[lb, pl.ds(k * L, L)]
                h = jnp.maximum(mvec - d, jnp.float32(0.0))
                h2 = h * h
                return (hs + h2,
                        hc + jnp.where(h2 > jnp.float32(0.0),
                                       jnp.float32(1.0), jnp.float32(0.0)))

            hs, hc = lax.fori_loop(0, TNP // L, hchunk, (zeros, zeros))
            part_v[pl.ds((1 + lb) * L, L)] = hs
            part_v[pl.ds((3 + lb) * L, L)] = hc

        pltpu.sync_copy(part_v, part_out.at[wid])

    return body


def kernel(outA, outB, matchA, matchB, nonMatchA, nonMatchB, hardNegative,
           device):
    B, N, D = outA.shape
    nM = matchA.shape[1]
    nNM = nonMatchA.shape[1]
    TM = (B * nM) // NW
    TMP = _cdiv(TM, CH) * CH
    TN = nNM // NS
    TNP = _cdiv(TN, CH) * CH
    n_local = B // NC

    tabA = outA.transpose(2, 0, 1).reshape(B * N * D)
    tabB = outB.transpose(2, 0, 1).reshape(B * N * D)
    offs = (jnp.arange(B, dtype=jnp.int32) * N)[:, None]
    c3 = jnp.arange(D, dtype=jnp.int32) * (B * N)

    def expand3(idx):
        return (idx[..., None] + c3).reshape(*idx.shape[:-1], -1)

    mA = expand3(jnp.pad((matchA.astype(jnp.int32) + offs).reshape(NW, TM),
                         ((0, 0), (0, TMP - TM))))
    mB = expand3(jnp.pad((matchB.astype(jnp.int32) + offs).reshape(NW, TM),
                         ((0, 0), (0, TMP - TM))))
    nmA = expand3(jnp.pad(
        (nonMatchA.astype(jnp.int32) + offs).reshape(B, NS, TN),
        ((0, 0), (0, 0), (0, TNP - TN))))
    nmB = expand3(jnp.pad(
        (nonMatchB.astype(jnp.int32) + offs).reshape(B, NS, TN),
        ((0, 0), (0, 0), (0, TNP - TN))))

    part = _make_kernel(B, nNM, TM, TMP, TN, TNP)(tabA, tabB, mA, mB,
                                                  nmA, nmB)

    matchLossSum = part[:, 0:L].sum() / nM
    pc = part.reshape(NC, NS, 8, L)
    nmSum = jnp.stack([pc[b // n_local, :, 1 + b % n_local, :].sum()
                       for b in range(B)])
    cnt = jnp.stack([pc[b // n_local, :, 3 + b % n_local, :].sum()
                     for b in range(B)])

    denom = jnp.where(cnt == 0, jnp.float32(nNM), cnt)
    hard = nmSum / denom
    soft = nmSum / nNM
    nmLoss = jnp.where(jnp.asarray(hardNegative) != 0, hard, soft)
    nonMatchLossSum = nmLoss.sum()
    contrastiveLossSum = matchLossSum + nonMatchLossSum
    return (contrastiveLossSum.astype(jnp.float32),
            matchLossSum.astype(jnp.float32),
            nonMatchLossSum.astype(jnp.float32))


# speedup vs baseline: 1.5612x; 1.5390x over previous
import functools

import jax
import jax.numpy as jnp
from jax import lax
from jax.experimental import pallas as pl
from jax.experimental.pallas import tpu as pltpu
from jax.experimental.pallas import tpu_sc as plsc

NC = 2
NS = 16
NW = NC * NS
L = 16
CH = 128
BIG = 1e30


def _cdiv(a, b):
    return (a + b - 1) // b


def _rsqrt_newton(s):
    i = plsc.bitcast(s, jnp.int32)
    i = jnp.int32(0x5F3759DF) - lax.shift_right_logical(i, 1)
    y = plsc.bitcast(i, jnp.float32)
    for _ in range(3):
        y = y * (jnp.float32(1.5) - jnp.float32(0.5) * s * y * y)
    return y


def _dist16(tA, tB, rows):
    r3 = rows * 3
    dx = plsc.load_gather(tA, [r3]) - plsc.load_gather(tB, [r3])
    dy = plsc.load_gather(tA, [r3 + 1]) - plsc.load_gather(tB, [r3 + 1])
    dz = plsc.load_gather(tA, [r3 + 2]) - plsc.load_gather(tB, [r3 + 2])
    return dx * dx + dy * dy + dz * dz


def _make_kernel(B, N, nM, nNM, TM, TMP, TN, TNP):
    mesh = plsc.VectorSubcoreMesh(core_axis_name="c", subcore_axis_name="s")
    n_mchunk = TMP // CH
    n_nchunk = TNP // CH
    n_local = B // NC
    TPB = NW // B
    BN = B * N

    @functools.partial(
        pl.kernel,
        mesh=mesh,
        compiler_params=pltpu.CompilerParams(use_tc_tiling_on_sc=False,
                                             needs_layout_passes=False),
        out_type=jax.ShapeDtypeStruct((NW, 8 * L), jnp.float32),
        scratch_types=[
            pltpu.VMEM((max(TMP, TNP),), jnp.int32),
            pltpu.VMEM((max(TMP, TNP),), jnp.int32),
            pltpu.VMEM((3 * max(TMP, TNP),), jnp.int32),
            pltpu.VMEM((3 * max(TMP, TNP),), jnp.int32),
            pltpu.VMEM((3 * CH,), jnp.float32),
            pltpu.VMEM((3 * CH,), jnp.float32),
            pltpu.VMEM((n_local, TNP), jnp.float32),
            pltpu.VMEM((n_local * L,), jnp.float32),
            pltpu.VMEM((NS, n_local * L), jnp.float32),
            pltpu.VMEM((8 * L,), jnp.float32),
            pltpu.VMEM_SHARED((NS, n_local * L), jnp.float32),
            pltpu.SemaphoreType.DMA,
            pltpu.SemaphoreType.DMA,
        ],
    )
    def body(tabA, tabB, mA, mB, nmA, nmB, part_out,
             rawA_v, rawB_v, idxA_v, idxB_v, rA, rB, dist_v, stage_v,
             allsums_v, part_v, shared, semA, semB):
        cid = lax.axis_index("c")
        sid = lax.axis_index("s")
        wid = cid * NS + sid
        lane = lax.iota(jnp.int32, L)
        zeros = jnp.zeros((L,), jnp.float32)
        zeros_i = jnp.zeros((L,), jnp.int32)

        def expand(c, boff, limit):
            for j in range(CH // L):
                pos = c * CH + j * L + lane
                valid = pos < limit
                p3 = pos * 3
                a = rawA_v[pl.ds(c * CH + j * L, L)] + boff
                b = rawB_v[pl.ds(c * CH + j * L, L)] + boff
                a = jnp.where(valid, a, zeros_i)
                b = jnp.where(valid, b, zeros_i)
                for t in range(3):
                    plsc.store_scatter(idxA_v, [p3 + t], a + t * BN)
                    plsc.store_scatter(idxB_v, [p3 + t], b + t * BN)

        def gather_chunk(c):
            cps = []
            for k in range(3):
                cps.append(pltpu.async_copy(
                    tabA.at[idxA_v.at[pl.ds((3 * c + k) * CH, CH)]],
                    rA.at[pl.ds(k * CH, CH)], semA))
                cps.append(pltpu.async_copy(
                    tabB.at[idxB_v.at[pl.ds((3 * c + k) * CH, CH)]],
                    rB.at[pl.ds(k * CH, CH)], semB))
            for cp in cps:
                cp.wait()

        pltpu.sync_copy(mA.at[wid], rawA_v.at[pl.ds(0, TMP)])
        pltpu.sync_copy(mB.at[wid], rawB_v.at[pl.ds(0, TMP)])

        def mchunk(c, acc):
            expand(c, 0, TM)
            gather_chunk(c)
            for j in range(CH // L):
                rows = j * L + lane
                s = _dist16(rA, rB, rows)
                valid = (c * CH + j * L + lane) < TM
                acc = acc + jnp.where(valid, s, jnp.float32(0.0))
            return acc

        macc = lax.fori_loop(0, n_mchunk, mchunk, zeros)
        part_v[pl.ds(0, L)] = macc
        for r in range(5, 8):
            part_v[pl.ds(r * L, L)] = zeros

        for lb in range(n_local):
            b = n_local * cid + lb
            pltpu.sync_copy(nmA.at[b, sid], rawA_v.at[pl.ds(0, TNP)])
            pltpu.sync_copy(nmB.at[b, sid], rawB_v.at[pl.ds(0, TNP)])

            def nchunk(c, acc):
                expand(c, 0, TN)
                gather_chunk(c)
                for j in range(CH // L):
                    rows = j * L + lane
                    s = _dist16(rA, rB, rows)
                    d = s * _rsqrt_newton(s)
                    d = jnp.where(s > jnp.float32(0.0), d, jnp.float32(0.0))
                    valid = (c * CH + j * L + lane) < TN
                    dist_v[lb, pl.ds(c * CH + j * L, L)] = jnp.where(
                        valid, d, jnp.float32(BIG))
                    acc = acc + jnp.where(valid, d, jnp.float32(0.0))
                return acc

            nacc = lax.fori_loop(0, n_nchunk, nchunk, zeros)
            stage_v[pl.ds(lb * L, L)] = nacc

        pltpu.sync_copy(stage_v, shared.at[sid])
        plsc.subcore_barrier()
        pltpu.sync_copy(shared, allsums_v)

        for lb in range(n_local):
            dacc = zeros
            for t in range(NS):
                dacc = dacc + allsums_v[t, pl.ds(lb * L, L)]
            mean = jnp.sum(dacc) * jnp.float32(1.0 / nNM)
            mvec = jnp.broadcast_to(mean, (L,))

            def hchunk(k, carry):
                hs, hc = carry
                d = dist_v[lb, pl.ds(k * L, L)]
                h = jnp.maximum(mvec - d, jnp.float32(0.0))
                h2 = h * h
                return (hs + h2,
                        hc + jnp.where(h2 > jnp.float32(0.0),
                                       jnp.float32(1.0), jnp.float32(0.0)))

            hs, hc = lax.fori_loop(0, TNP // L, hchunk, (zeros, zeros))
            part_v[pl.ds((1 + lb) * L, L)] = hs
            part_v[pl.ds((3 + lb) * L, L)] = hc

        pltpu.sync_copy(part_v, part_out.at[wid])

    return body


def kernel(outA, outB, matchA, matchB, nonMatchA, nonMatchB, hardNegative,
           device):
    B, N, D = outA.shape
    nM = matchA.shape[1]
    nNM = nonMatchA.shape[1]
    TM = (B * nM) // NW
    TMP = _cdiv(TM, CH) * CH
    TN = nNM // NS
    TNP = _cdiv(TN, CH) * CH

    tabA = outA.transpose(2, 0, 1).reshape(B * N * D)
    tabB = outB.transpose(2, 0, 1).reshape(B * N * D)

    offs = (jnp.arange(B, dtype=jnp.int32) * N)[:, None]
    mAp = jnp.pad((matchA.astype(jnp.int32) + offs).reshape(NW, TM),
                  ((0, 0), (0, TMP - TM)))
    mBp = jnp.pad((matchB.astype(jnp.int32) + offs).reshape(NW, TM),
                  ((0, 0), (0, TMP - TM)))
    nmAp = jnp.pad((nonMatchA.astype(jnp.int32) + offs).reshape(B, NS, TN),
                   ((0, 0), (0, 0), (0, TNP - TN)))
    nmBp = jnp.pad((nonMatchB.astype(jnp.int32) + offs).reshape(B, NS, TN),
                   ((0, 0), (0, 0), (0, TNP - TN)))

    part = _make_kernel(B, N, nM, nNM, TM, TMP, TN, TNP)(
        tabA, tabB, mAp, mBp, nmAp, nmBp)

    matchLossSum = part[:, 0:L].sum() / nM
    n_local = B // NC
    pc = part.reshape(NC, NS, 8, L)
    nmSum = jnp.stack([pc[b // n_local, :, 1 + b % n_local, :].sum()
                       for b in range(B)])
    cnt = jnp.stack([pc[b // n_local, :, 3 + b % n_local, :].sum()
                     for b in range(B)])

    denom = jnp.where(cnt == 0, jnp.float32(nNM), cnt)
    hard = nmSum / denom
    soft = nmSum / nNM
    nmLoss = jnp.where(jnp.asarray(hardNegative) != 0, hard, soft)
    nonMatchLossSum = nmLoss.sum()
    contrastiveLossSum = matchLossSum + nonMatchLossSum
    return (contrastiveLossSum.astype(jnp.float32),
            matchLossSum.astype(jnp.float32),
            nonMatchLossSum.astype(jnp.float32))


# speedup vs baseline: 1.5800x; 1.0121x over previous
import functools

import jax
import jax.numpy as jnp
from jax import lax
from jax.experimental import pallas as pl
from jax.experimental.pallas import tpu as pltpu
from jax.experimental.pallas import tpu_sc as plsc

NC = 2
NS = 16
NW = NC * NS
L = 16
CH = 128
BIG = 1e30


def _cdiv(a, b):
    return (a + b - 1) // b


def _rsqrt_newton(s):
    i = plsc.bitcast(s, jnp.int32)
    i = jnp.int32(0x5F3759DF) - lax.shift_right_logical(i, 1)
    y = plsc.bitcast(i, jnp.float32)
    for _ in range(3):
        y = y * (jnp.float32(1.5) - jnp.float32(0.5) * s * y * y)
    return y


def _dist16(tA, tB, rows):
    r3 = rows * 3
    dx = plsc.load_gather(tA, [r3]) - plsc.load_gather(tB, [r3])
    dy = plsc.load_gather(tA, [r3 + 1]) - plsc.load_gather(tB, [r3 + 1])
    dz = plsc.load_gather(tA, [r3 + 2]) - plsc.load_gather(tB, [r3 + 2])
    return dx * dx + dy * dy + dz * dz


def _make_kernel(B, N, nM, nNM, TM, TMP, TN, TNP):
    mesh = plsc.VectorSubcoreMesh(core_axis_name="c", subcore_axis_name="s")
    n_mchunk = TMP // CH
    n_nchunk = TNP // CH
    n_local = B // NC
    BN = B * N
    SEG = N // NS

    @functools.partial(
        pl.kernel,
        mesh=mesh,
        compiler_params=pltpu.CompilerParams(use_tc_tiling_on_sc=False,
                                             needs_layout_passes=False),
        out_type=jax.ShapeDtypeStruct((NW, 8 * L), jnp.float32),
        scratch_types=[
            pltpu.VMEM((max(TMP, TNP),), jnp.int32),
            pltpu.VMEM((max(TMP, TNP),), jnp.int32),
            pltpu.VMEM((3 * max(TMP, TNP),), jnp.int32),
            pltpu.VMEM((3 * max(TMP, TNP),), jnp.int32),
            pltpu.VMEM((3 * CH,), jnp.float32),
            pltpu.VMEM((3 * CH,), jnp.float32),
            pltpu.VMEM((n_local, TNP), jnp.float32),
            pltpu.VMEM((n_local * L,), jnp.float32),
            pltpu.VMEM((NS, n_local * L), jnp.float32),
            pltpu.VMEM((8 * L,), jnp.float32),
            pltpu.VMEM_SHARED((3 * N,), jnp.float32),
            pltpu.VMEM_SHARED((NS, n_local * L), jnp.float32),
            pltpu.SemaphoreType.DMA,
            pltpu.SemaphoreType.DMA,
        ],
    )
    def body(tabA, tabB, mA, mB, nmA, nmB, part_out,
             rawA_v, rawB_v, idxA_v, idxB_v, rA, rB, dist_v, stage_v,
             allsums_v, part_v, shtab, shared, semA, semB):
        cid = lax.axis_index("c")
        sid = lax.axis_index("s")
        wid = cid * NS + sid
        lane = lax.iota(jnp.int32, L)
        zeros = jnp.zeros((L,), jnp.float32)
        zeros_i = jnp.zeros((L,), jnp.int32)

        def expand(c, limit, boff):
            for j in range(CH // L):
                pos = c * CH + j * L + lane
                valid = pos < limit
                p3 = pos * 3
                a = rawA_v[pl.ds(c * CH + j * L, L)]
                b = rawB_v[pl.ds(c * CH + j * L, L)]
                a = jnp.where(valid, a, zeros_i)
                b = jnp.where(valid, b + boff, zeros_i)
                for t in range(3):
                    plsc.store_scatter(idxA_v, [p3 + t], a + t * N)
                    plsc.store_scatter(idxB_v, [p3 + t], b + t * BN)

        def gather_chunk(c):
            cps = []
            for k in range(3):
                cps.append(pltpu.async_copy(
                    shtab.at[idxA_v.at[pl.ds((3 * c + k) * CH, CH)]],
                    rA.at[pl.ds(k * CH, CH)], semA))
                cps.append(pltpu.async_copy(
                    tabB.at[idxB_v.at[pl.ds((3 * c + k) * CH, CH)]],
                    rB.at[pl.ds(k * CH, CH)], semB))
            for cp in cps:
                cp.wait()

        part_v[pl.ds(0, L)] = zeros
        for r in range(5, 8):
            part_v[pl.ds(r * L, L)] = zeros

        macc = zeros
        for lb in range(n_local):
            b = n_local * cid + lb

            for comp in range(3):
                pltpu.sync_copy(
                    tabA.at[pl.ds(comp * BN + b * N + sid * SEG, SEG)],
                    shtab.at[pl.ds(comp * N + sid * SEG, SEG)])
            plsc.subcore_barrier()

            pltpu.sync_copy(mA.at[b, sid], rawA_v.at[pl.ds(0, TMP)])
            pltpu.sync_copy(mB.at[b, sid], rawB_v.at[pl.ds(0, TMP)])

            def mchunk(c, acc):
                expand(c, TM, b * N)
                gather_chunk(c)
                for j in range(CH // L):
                    rows = j * L + lane
                    s = _dist16(rA, rB, rows)
                    valid = (c * CH + j * L + lane) < TM
                    acc = acc + jnp.where(valid, s, jnp.float32(0.0))
                return acc

            macc = lax.fori_loop(0, n_mchunk, mchunk, macc)

            pltpu.sync_copy(nmA.at[b, sid], rawA_v.at[pl.ds(0, TNP)])
            pltpu.sync_copy(nmB.at[b, sid], rawB_v.at[pl.ds(0, TNP)])

            def nchunk(c, acc):
                expand(c, TN, b * N)
                gather_chunk(c)
                for j in range(CH // L):
                    rows = j * L + lane
                    s = _dist16(rA, rB, rows)
                    d = s * _rsqrt_newton(s)
                    d = jnp.where(s > jnp.float32(0.0), d, jnp.float32(0.0))
                    valid = (c * CH + j * L + lane) < TN
                    dist_v[lb, pl.ds(c * CH + j * L, L)] = jnp.where(
                        valid, d, jnp.float32(BIG))
                    acc = acc + jnp.where(valid, d, jnp.float32(0.0))
                return acc

            nacc = lax.fori_loop(0, n_nchunk, nchunk, zeros)
            stage_v[pl.ds(lb * L, L)] = nacc
            plsc.subcore_barrier()

        part_v[pl.ds(0, L)] = macc

        pltpu.sync_copy(stage_v, shared.at[sid])
        plsc.subcore_barrier()
        pltpu.sync_copy(shared, allsums_v)

        for lb in range(n_local):
            dacc = zeros
            for t in range(NS):
                dacc = dacc + allsums_v[t, pl.ds(lb * L, L)]
            mean = jnp.sum(dacc) * jnp.float32(1.0 / nNM)
            mvec = jnp.broadcast_to(mean, (L,))

            def hchunk(k, carry):
                hs, hc = carry
                d = dist_v[lb, pl.ds(k * L, L)]
                h = jnp.maximum(mvec - d, jnp.float32(0.0))
                h2 = h * h
                return (hs + h2,
                        hc + jnp.where(h2 > jnp.float32(0.0),
                                       jnp.float32(1.0), jnp.float32(0.0)))

            hs, hc = lax.fori_loop(0, TNP // L, hchunk, (zeros, zeros))
            part_v[pl.ds((1 + lb) * L, L)] = hs
            part_v[pl.ds((3 + lb) * L, L)] = hc

        pltpu.sync_copy(part_v, part_out.at[wid])

    return body


def kernel(outA, outB, matchA, matchB, nonMatchA, nonMatchB, hardNegative,
           device):
    B, N, D = outA.shape
    nM = matchA.shape[1]
    nNM = nonMatchA.shape[1]
    TM = nM // NS
    TMP = _cdiv(TM, CH) * CH
    TN = nNM // NS
    TNP = _cdiv(TN, CH) * CH

    tabA = outA.transpose(2, 0, 1).reshape(B * N * D)
    tabB = outB.transpose(2, 0, 1).reshape(B * N * D)

    mAp = jnp.pad(matchA.astype(jnp.int32).reshape(B, NS, TM),
                  ((0, 0), (0, 0), (0, TMP - TM)))
    mBp = jnp.pad(matchB.astype(jnp.int32).reshape(B, NS, TM),
                  ((0, 0), (0, 0), (0, TMP - TM)))
    nmAp = jnp.pad(nonMatchA.astype(jnp.int32).reshape(B, NS, TN),
                   ((0, 0), (0, 0), (0, TNP - TN)))
    nmBp = jnp.pad(nonMatchB.astype(jnp.int32).reshape(B, NS, TN),
                   ((0, 0), (0, 0), (0, TNP - TN)))

    part = _make_kernel(B, N, nM, nNM, TM, TMP, TN, TNP)(
        tabA, tabB, mAp, mBp, nmAp, nmBp)

    matchLossSum = part[:, 0:L].sum() / nM
    n_local = B // NC
    pc = part.reshape(NC, NS, 8, L)
    nmSum = jnp.stack([pc[b // n_local, :, 1 + b % n_local, :].sum()
                       for b in range(B)])
    cnt = jnp.stack([pc[b // n_local, :, 3 + b % n_local, :].sum()
                     for b in range(B)])

    denom = jnp.where(cnt == 0, jnp.float32(nNM), cnt)
    hard = nmSum / denom
    soft = nmSum / nNM
    nmLoss = jnp.where(jnp.asarray(hardNegative) != 0, hard, soft)
    nonMatchLossSum = nmLoss.sum()
    contrastiveLossSum = matchLossSum + nonMatchLossSum
    return (contrastiveLossSum.astype(jnp.float32),
            matchLossSum.astype(jnp.float32),
            nonMatchLossSum.astype(jnp.float32))
